# Initial kernel scaffold; baseline (speedup 1.0000x reference)
import jax
import jax.numpy as jnp
from jax.experimental import pallas as pl


def kernel(inputs, table):
    raise NotImplementedError("write your pallas kernel here")


# speedup vs baseline: 1.9700x; 1.9700x over previous
import functools

import jax
import jax.numpy as jnp
from jax import lax
from jax.experimental import pallas as pl
from jax.experimental.pallas import tpu as pltpu
from jax.experimental.pallas import tpu_sc as plsc

_VOCAB = 8
_EMBED = 2
_LANES = 16


@functools.lru_cache(maxsize=None)
def _build_embed_kernel(batch: int):
    info = plsc.get_sparse_core_info()
    nc, ns = info.num_cores, info.num_subcores
    nw = nc * ns
    assert batch % (nw * _LANES) == 0
    b_per_w = batch // nw
    mesh = plsc.VectorSubcoreMesh(core_axis_name="c", subcore_axis_name="s")

    @functools.partial(
        pl.kernel,
        out_type=jax.ShapeDtypeStruct((batch * _EMBED,), jnp.float32),
        mesh=mesh,
        scratch_types=[
            pltpu.VMEM((b_per_w,), jnp.int32),
            pltpu.VMEM((_VOCAB * _EMBED,), jnp.float32),
            pltpu.VMEM((b_per_w * _EMBED,), jnp.float32),
        ],
        compiler_params=pltpu.CompilerParams(needs_layout_passes=False),
    )
    def embed(ids_hbm, table_hbm, out_hbm, idx_v, tab_v, out_v):
        wid = lax.axis_index("s") * nc + lax.axis_index("c")
        base = wid * b_per_w
        pltpu.sync_copy(table_hbm, tab_v)
        pltpu.sync_copy(ids_hbm.at[pl.ds(base, b_per_w)], idx_v)
        lane = lax.iota(jnp.int32, _LANES)
        for j in range(b_per_w // _LANES):
            ids = idx_v[pl.ds(j * _LANES, _LANES)]
            flat = ids * _EMBED
            c0 = plsc.load_gather(tab_v, [flat])
            c1 = plsc.load_gather(tab_v, [flat + 1])
            pos = (lane + j * _LANES) * _EMBED
            plsc.store_scatter(out_v, [pos], c0)
            plsc.store_scatter(out_v, [pos + 1], c1)
        pltpu.sync_copy(out_v, out_hbm.at[pl.ds(base * _EMBED, b_per_w * _EMBED)])

    return embed


def kernel(inputs, table):
    batch = inputs.shape[0]
    ids = inputs.reshape(batch)
    flat_table = table.astype(jnp.float32).reshape(_VOCAB * _EMBED)
    out = _build_embed_kernel(batch)(ids, flat_table)
    return out.reshape(batch, _EMBED)


# speedup vs baseline: 2.0180x; 1.0244x over previous
import functools

import jax
import jax.numpy as jnp
from jax import lax
from jax.experimental import pallas as pl
from jax.experimental.pallas import tpu as pltpu
from jax.experimental.pallas import tpu_sc as plsc

_VOCAB = 8
_EMBED = 2
_LANES = 16


@functools.lru_cache(maxsize=None)
def _build_embed_kernel(batch: int):
    info = plsc.get_sparse_core_info()
    nc, ns = info.num_cores, info.num_subcores
    nw = nc * ns
    assert batch % (nw * _LANES) == 0
    b_per_w = batch // nw
    mesh = plsc.VectorSubcoreMesh(core_axis_name="c", subcore_axis_name="s")

    @functools.partial(
        pl.kernel,
        out_type=jax.ShapeDtypeStruct((batch * _EMBED,), jnp.float32),
        mesh=mesh,
        scratch_types=[
            pltpu.VMEM((b_per_w,), jnp.int32),
            pltpu.VMEM((_VOCAB * _EMBED,), jnp.float32),
            pltpu.VMEM((b_per_w * _EMBED,), jnp.float32),
            pltpu.SemaphoreType.DMA,
            pltpu.SemaphoreType.DMA,
        ],
        compiler_params=pltpu.CompilerParams(needs_layout_passes=False),
    )
    def embed(ids_hbm, table_hbm, out_hbm, idx_v, tab_v, out_v, sem_i, sem_t):
        wid = lax.axis_index("s") * nc + lax.axis_index("c")
        base = wid * b_per_w
        cp_ids = pltpu.async_copy(ids_hbm.at[pl.ds(base, b_per_w)], idx_v, sem_i)
        cp_tab = pltpu.async_copy(table_hbm, tab_v, sem_t)
        cp_ids.wait()
        cp_tab.wait()
        lane = lax.iota(jnp.int32, _LANES)

        def body(j, carry):
            ids = idx_v[pl.ds(j * _LANES, _LANES)]
            flat = ids * _EMBED
            c0 = plsc.load_gather(tab_v, [flat])
            c1 = plsc.load_gather(tab_v, [flat + 1])
            pos = (lane + j * _LANES) * _EMBED
            plsc.store_scatter(out_v, [pos], c0)
            plsc.store_scatter(out_v, [pos + 1], c1)
            return carry

        lax.fori_loop(0, b_per_w // _LANES, body, 0)
        pltpu.sync_copy(out_v, out_hbm.at[pl.ds(base * _EMBED, b_per_w * _EMBED)])

    return embed


def kernel(inputs, table):
    batch = inputs.shape[0]
    ids = inputs.reshape(batch)
    flat_table = table.astype(jnp.float32).reshape(_VOCAB * _EMBED)
    out = _build_embed_kernel(batch)(ids, flat_table)
    return out.reshape(batch, _EMBED)


# speedup vs baseline: 2.0905x; 1.0359x over previous
import functools

import jax
import jax.numpy as jnp
from jax import lax
from jax.experimental import pallas as pl
from jax.experimental.pallas import tpu as pltpu
from jax.experimental.pallas import tpu_sc as plsc

_VOCAB = 8
_EMBED = 2
_LANES = 16


@functools.lru_cache(maxsize=None)
def _build_embed_kernel(batch: int):
    info = plsc.get_sparse_core_info()
    nc, ns = info.num_cores, info.num_subcores
    nw = nc * ns
    assert batch % (nw * _LANES) == 0
    b_per_w = batch // nw
    mesh = plsc.VectorSubcoreMesh(
        core_axis_name="c", subcore_axis_name="s", num_cores=1
    )
    nc = 1
    nw = nc * ns
    b_per_w = batch // nw

    @functools.partial(
        pl.kernel,
        out_type=jax.ShapeDtypeStruct((batch * _EMBED,), jnp.float32),
        mesh=mesh,
        scratch_types=[
            pltpu.VMEM((b_per_w,), jnp.int32),
            pltpu.VMEM((_VOCAB * _EMBED,), jnp.float32),
            pltpu.VMEM((b_per_w * _EMBED,), jnp.float32),
            pltpu.SemaphoreType.DMA,
            pltpu.SemaphoreType.DMA,
        ],
        compiler_params=pltpu.CompilerParams(needs_layout_passes=False),
    )
    def embed(ids_hbm, table_hbm, out_hbm, idx_v, tab_v, out_v, sem_i, sem_t):
        wid = lax.axis_index("s") * nc + lax.axis_index("c")
        base = wid * b_per_w
        cp_ids = pltpu.async_copy(ids_hbm.at[pl.ds(base, b_per_w)], idx_v, sem_i)
        cp_tab = pltpu.async_copy(table_hbm, tab_v, sem_t)
        cp_ids.wait()
        cp_tab.wait()
        lane = lax.iota(jnp.int32, _LANES)

        def body(j, carry):
            ids = idx_v[pl.ds(j * _LANES, _LANES)]
            flat = ids * _EMBED
            c0 = plsc.load_gather(tab_v, [flat])
            c1 = plsc.load_gather(tab_v, [flat + 1])
            pos = (lane + j * _LANES) * _EMBED
            plsc.store_scatter(out_v, [pos], c0)
            plsc.store_scatter(out_v, [pos + 1], c1)
            return carry

        lax.fori_loop(0, b_per_w // _LANES, body, 0)
        pltpu.sync_copy(out_v, out_hbm.at[pl.ds(base * _EMBED, b_per_w * _EMBED)])

    return embed


def kernel(inputs, table):
    batch = inputs.shape[0]
    ids = inputs.reshape(batch)
    flat_table = table.astype(jnp.float32).reshape(_VOCAB * _EMBED)
    out = _build_embed_kernel(batch)(ids, flat_table)
    return out.reshape(batch, _EMBED)


# speedup vs baseline: 2.0967x; 1.0030x over previous
import functools

import jax
import jax.numpy as jnp
from jax import lax
from jax.experimental import pallas as pl
from jax.experimental.pallas import tpu as pltpu
from jax.experimental.pallas import tpu_sc as plsc

_VOCAB = 8
_EMBED = 2
_LANES = 16


@functools.lru_cache(maxsize=None)
def _build_embed_kernel(batch: int):
    info = plsc.get_sparse_core_info()
    nc, ns = info.num_cores, info.num_subcores
    nw = nc * ns
    assert batch % (nw * _LANES) == 0
    b_per_w = batch // nw
    mesh = plsc.VectorSubcoreMesh(
        core_axis_name="c", subcore_axis_name="s", num_cores=1
    )
    nc = 1
    nw = nc * ns
    b_per_w = batch // nw

    @functools.partial(
        pl.kernel,
        out_type=jax.ShapeDtypeStruct((batch * _EMBED,), jnp.float32),
        mesh=mesh,
        scratch_types=[
            pltpu.VMEM((b_per_w,), jnp.int32),
            pltpu.VMEM((_VOCAB * _EMBED,), jnp.float32),
            pltpu.VMEM((b_per_w * _EMBED,), jnp.float32),
            pltpu.SemaphoreType.DMA,
            pltpu.SemaphoreType.DMA,
        ],
        compiler_params=pltpu.CompilerParams(
            needs_layout_passes=False, skip_device_barrier=True
        ),
    )
    def embed(ids_hbm, table_hbm, out_hbm, idx_v, tab_v, out_v, sem_i, sem_t):
        wid = lax.axis_index("s") * nc + lax.axis_index("c")
        base = wid * b_per_w
        cp_ids = pltpu.async_copy(ids_hbm.at[pl.ds(base, b_per_w)], idx_v, sem_i)
        cp_tab = pltpu.async_copy(table_hbm, tab_v, sem_t)
        cp_ids.wait()
        cp_tab.wait()
        lane = lax.iota(jnp.int32, _LANES)

        def body(j, carry):
            ids = idx_v[pl.ds(j * _LANES, _LANES)]
            flat = ids * _EMBED
            c0 = plsc.load_gather(tab_v, [flat])
            c1 = plsc.load_gather(tab_v, [flat + 1])
            pos = (lane + j * _LANES) * _EMBED
            plsc.store_scatter(out_v, [pos], c0)
            plsc.store_scatter(out_v, [pos + 1], c1)
            return carry

        lax.fori_loop(0, b_per_w // _LANES, body, 0)
        pltpu.sync_copy(out_v, out_hbm.at[pl.ds(base * _EMBED, b_per_w * _EMBED)])

    return embed


def kernel(inputs, table):
    batch = inputs.shape[0]
    ids = inputs.reshape(batch)
    flat_table = table.astype(jnp.float32).reshape(_VOCAB * _EMBED)
    out = _build_embed_kernel(batch)(ids, flat_table)
    return out.reshape(batch, _EMBED)
